# Initial kernel scaffold; baseline (speedup 1.0000x reference)
import jax
import jax.numpy as jnp
from jax.experimental import pallas as pl


def kernel(x, edge_index, W1, b1, W2, b2, Wl, bl):
    raise NotImplementedError("write your pallas kernel here")


# speedup vs baseline: 21.2229x; 21.2229x over previous
import functools

import jax
import jax.numpy as jnp
from jax import lax
from jax.experimental import pallas as pl
from jax.experimental.pallas import tpu as pltpu
from jax.experimental.pallas import tpu_sc as plsc

NC = 2
NS = 16
NW = NC * NS
K = 80


def _flat_wid():
    return lax.axis_index("s") * NC + lax.axis_index("c")


def _make_deg_kernel(n_pad, nchunk):
    mesh = plsc.VectorSubcoreMesh(core_axis_name="c", subcore_axis_name="s")
    rps = n_pad // NS

    @functools.partial(
        pl.kernel,
        out_type=jax.ShapeDtypeStruct((NC, n_pad), jnp.float32),
        mesh=mesh,
        scratch_types=[
            pltpu.VMEM_SHARED((n_pad,), jnp.float32),
            pltpu.VMEM((nchunk, K), jnp.int32),
            pltpu.VMEM((K,), jnp.float32),
            pltpu.VMEM((rps,), jnp.float32),
        ],
    )
    def k(dst_hbm, out_hbm, acc, dstv, ones_v, zbuf):
        c = lax.axis_index("c")
        s = lax.axis_index("s")
        wid = _flat_wid()

        for i in range(K // 16):
            ones_v[pl.ds(16 * i, 16)] = jnp.ones((16,), jnp.float32)

        def zfill(i, _):
            zbuf[pl.ds(16 * i, 16)] = jnp.zeros((16,), jnp.float32)
            return 0
        lax.fori_loop(0, rps // 16, zfill, 0)
        pltpu.sync_copy(zbuf, acc.at[pl.ds(s * rps, rps)])

        pltpu.sync_copy(dst_hbm.at[wid], dstv)
        plsc.subcore_barrier()

        def body(ci, _):
            pltpu.sync_copy(ones_v, acc.at[dstv.at[ci]], add=True)
            return 0
        lax.fori_loop(0, nchunk, body, 0)

        plsc.subcore_barrier()
        pltpu.sync_copy(acc.at[pl.ds(s * rps, rps)],
                        out_hbm.at[c, pl.ds(s * rps, rps)])

    return k


def _make_agg_kernel(n_pad, nchunk, d):
    mesh = plsc.VectorSubcoreMesh(core_axis_name="c", subcore_axis_name="s")
    rps = n_pad // NS

    @functools.partial(
        pl.kernel,
        out_type=jax.ShapeDtypeStruct((NC, n_pad, d), jnp.float32),
        mesh=mesh,
        scratch_types=[
            pltpu.VMEM_SHARED((n_pad, d), jnp.float32),
            pltpu.VMEM((nchunk, K), jnp.int32),
            pltpu.VMEM((nchunk, K), jnp.int32),
            pltpu.VMEM((K, d), jnp.float32),
            pltpu.SemaphoreType.DMA,
        ],
        compiler_params=pltpu.CompilerParams(use_tc_tiling_on_sc=False),
    )
    def k(y_hbm, src_hbm, dst_hbm, out_hbm, acc, srcv, dstv, rows, sem):
        c = lax.axis_index("c")
        s = lax.axis_index("s")
        wid = _flat_wid()

        def zfill(i, _):
            for j in range(d // 16):
                rows[i, pl.ds(16 * j, 16)] = jnp.zeros((16,), jnp.float32)
            return 0
        lax.fori_loop(0, K, zfill, 0)
        for t in range(rps // K):
            pltpu.sync_copy(rows, acc.at[pl.ds(s * rps + t * K, K), :])

        pltpu.sync_copy(src_hbm.at[wid], srcv)
        pltpu.sync_copy(dst_hbm.at[wid], dstv)
        plsc.subcore_barrier()

        def body(ci, _):
            pltpu.async_copy(y_hbm.at[srcv.at[ci]], rows, sem).wait()
            pltpu.sync_copy(rows, acc.at[dstv.at[ci]], add=True)
            return 0
        lax.fori_loop(0, nchunk, body, 0)

        plsc.subcore_barrier()
        pltpu.sync_copy(acc.at[pl.ds(s * rps, rps), :],
                        out_hbm.at[c, pl.ds(s * rps, rps), :])

    return k


def _mm(a, b):
    return jnp.dot(a, b, preferred_element_type=jnp.float32,
                   precision=lax.Precision.HIGHEST)


def _tc1_body(x_ref, w1_ref, d0_ref, d1_ref, dinv_ref, y1_ref):
    dinv = lax.rsqrt(d0_ref[...] + d1_ref[...] + 1.0)
    dinv_ref[...] = dinv
    y1_ref[...] = _mm(x_ref[...], w1_ref[...]) * dinv


def _tc2_body(s1a_ref, s1b_ref, y1_ref, dinv_ref, b1_ref, w2_ref, y2_ref):
    dinv = dinv_ref[...]
    h1 = jnp.maximum(dinv * (s1a_ref[...] + s1b_ref[...] + y1_ref[...])
                     + b1_ref[...], 0.0)
    y2_ref[...] = _mm(h1, w2_ref[...]) * dinv


def _tc3_body(s2a_ref, s2b_ref, y2_ref, dinv_ref, b2_ref, wlt_ref, bl_ref,
              h2_ref, out_ref):
    dinv = dinv_ref[...]
    h2 = jnp.maximum(dinv * (s2a_ref[...] + s2b_ref[...] + y2_ref[...])
                     + b2_ref[...], 0.0)
    h2_ref[...] = h2
    out_ref[...] = _mm(h2, wlt_ref[...]) + bl_ref[...]


def kernel(x, edge_index, W1, b1, W2, b2, Wl, bl):
    n, d_in = x.shape
    d_hid = W1.shape[1]
    d_out = W2.shape[1]
    e = edge_index.shape[1]
    epw = e // NW
    nchunk = epw // K
    n_pad = ((n + NS * K - 1) // (NS * K)) * (NS * K)

    src3 = edge_index[0].reshape(NW, nchunk, K)
    dst3 = edge_index[1].reshape(NW, nchunk, K)

    deg_p = _make_deg_kernel(n_pad, nchunk)(dst3)
    deg0 = deg_p[0, :n].reshape(n, 1)
    deg1 = deg_p[1, :n].reshape(n, 1)

    dinv, y1 = pl.pallas_call(
        _tc1_body,
        out_shape=[jax.ShapeDtypeStruct((n, 1), jnp.float32),
                   jax.ShapeDtypeStruct((n, d_hid), jnp.float32)],
    )(x, W1, deg0, deg1)

    s1_p = _make_agg_kernel(n_pad, nchunk, d_hid)(y1, src3, dst3)

    y2 = pl.pallas_call(
        _tc2_body,
        out_shape=jax.ShapeDtypeStruct((n, d_out), jnp.float32),
    )(s1_p[0, :n], s1_p[1, :n], y1, dinv, b1.reshape(1, d_hid), W2)

    s2_p = _make_agg_kernel(n_pad, nchunk, d_out)(y2, src3, dst3)

    h2, out = pl.pallas_call(
        _tc3_body,
        out_shape=[jax.ShapeDtypeStruct((n, d_out), jnp.float32),
                   jax.ShapeDtypeStruct((n, d_out), jnp.float32)],
    )(s2_p[0, :n], s2_p[1, :n], y2, dinv, b2.reshape(1, d_out), Wl.T,
      bl.reshape(1, d_out))

    return (h2, out)


# speedup vs baseline: 30.2714x; 1.4264x over previous
import functools

import jax
import jax.numpy as jnp
from jax import lax
from jax.experimental import pallas as pl
from jax.experimental.pallas import tpu as pltpu
from jax.experimental.pallas import tpu_sc as plsc

NC = 2
NS = 16
NW = NC * NS
K = 80


def _flat_wid():
    return lax.axis_index("s") * NC + lax.axis_index("c")


def _make_deg_kernel(n_pad, nchunk):
    mesh = plsc.VectorSubcoreMesh(core_axis_name="c", subcore_axis_name="s")
    rps = n_pad // NS

    @functools.partial(
        pl.kernel,
        out_type=jax.ShapeDtypeStruct((NC, n_pad), jnp.float32),
        mesh=mesh,
        scratch_types=[
            pltpu.VMEM_SHARED((n_pad,), jnp.float32),
            pltpu.VMEM((nchunk, K), jnp.int32),
            pltpu.VMEM((K,), jnp.float32),
            pltpu.VMEM((rps,), jnp.float32),
        ],
    )
    def k(dst_hbm, out_hbm, acc, dstv, ones_v, zbuf):
        c = lax.axis_index("c")
        s = lax.axis_index("s")
        wid = _flat_wid()

        for i in range(K // 16):
            ones_v[pl.ds(16 * i, 16)] = jnp.ones((16,), jnp.float32)

        def zfill(i, _):
            zbuf[pl.ds(16 * i, 16)] = jnp.zeros((16,), jnp.float32)
            return 0
        lax.fori_loop(0, rps // 16, zfill, 0)
        pltpu.sync_copy(zbuf, acc.at[pl.ds(s * rps, rps)])

        pltpu.sync_copy(dst_hbm.at[wid], dstv)
        plsc.subcore_barrier()

        def body(ci, _):
            pltpu.sync_copy(ones_v, acc.at[dstv.at[ci]], add=True)
            return 0
        lax.fori_loop(0, nchunk, body, 0)

        plsc.subcore_barrier()
        pltpu.sync_copy(acc.at[pl.ds(s * rps, rps)],
                        out_hbm.at[c, pl.ds(s * rps, rps)])

    return k


def _make_agg_kernel(n_pad, nchunk, d, nblk):
    mesh = plsc.VectorSubcoreMesh(core_axis_name="c", subcore_axis_name="s")
    rps = n_pad // NS
    cpb = nchunk // nblk

    @functools.partial(
        pl.kernel,
        out_type=jax.ShapeDtypeStruct((NC, n_pad, d), jnp.float32),
        mesh=mesh,
        scratch_types=[
            pltpu.VMEM_SHARED((n_pad, d), jnp.float32),
            pltpu.VMEM((cpb, K), jnp.int32),
            pltpu.VMEM((cpb, K), jnp.int32),
            pltpu.VMEM((K, d), jnp.float32),
            pltpu.VMEM((K, d), jnp.float32),
            pltpu.SemaphoreType.DMA,
            pltpu.SemaphoreType.DMA,
        ],
        compiler_params=pltpu.CompilerParams(use_tc_tiling_on_sc=False),
    )
    def k(y_hbm, src_hbm, dst_hbm, out_hbm, acc, srcv, dstv, rows0, rows1,
          gsem0, gsem1):
        c = lax.axis_index("c")
        s = lax.axis_index("s")
        wid = _flat_wid()
        bufs = ((rows0, gsem0), (rows1, gsem1))

        def zfill(i, _):
            for j in range(d // 16):
                rows0[i, pl.ds(16 * j, 16)] = jnp.zeros((16,), jnp.float32)
            return 0
        lax.fori_loop(0, K, zfill, 0)
        for t in range(rps // K):
            pltpu.sync_copy(rows0, acc.at[pl.ds(s * rps + t * K, K), :])
        plsc.subcore_barrier()

        npairs = cpb // 2
        for blk in range(nblk):
            pltpu.sync_copy(src_hbm.at[wid, pl.ds(blk * cpb, cpb), :], srcv)
            pltpu.sync_copy(dst_hbm.at[wid, pl.ds(blk * cpb, cpb), :], dstv)
            pltpu.async_copy(y_hbm.at[srcv.at[0]], rows0, gsem0)
            pltpu.async_copy(y_hbm.at[srcv.at[1]], rows1, gsem1)

            def body(g, _):
                for b, (rb, gs) in enumerate(bufs):
                    ci = 2 * g + b
                    pltpu.make_async_copy(y_hbm.at[srcv.at[ci]], rb, gs).wait()
                    pltpu.sync_copy(rb, acc.at[dstv.at[ci]], add=True)
                    pltpu.async_copy(y_hbm.at[srcv.at[ci + 2]], rb, gs)
                return 0
            lax.fori_loop(0, npairs - 1, body, 0)

            for b, (rb, gs) in enumerate(bufs):
                ci = 2 * (npairs - 1) + b
                pltpu.make_async_copy(y_hbm.at[srcv.at[ci]], rb, gs).wait()
                pltpu.sync_copy(rb, acc.at[dstv.at[ci]], add=True)
                if ci + 2 < cpb:
                    pltpu.async_copy(y_hbm.at[srcv.at[ci + 2]], rb, gs)
            if cpb % 2:
                ci = cpb - 1
                rb, gs = bufs[ci % 2]
                pltpu.make_async_copy(y_hbm.at[srcv.at[ci]], rb, gs).wait()
                pltpu.sync_copy(rb, acc.at[dstv.at[ci]], add=True)

        plsc.subcore_barrier()
        pltpu.sync_copy(acc.at[pl.ds(s * rps, rps), :],
                        out_hbm.at[c, pl.ds(s * rps, rps), :])

    return k


def _mm(a, b):
    return jnp.dot(a, b, preferred_element_type=jnp.float32,
                   precision=lax.Precision.HIGHEST)


def _tc1_body(x_ref, w1_ref, d0_ref, d1_ref, dinv_ref, y1_ref):
    dinv = lax.rsqrt(d0_ref[...] + d1_ref[...] + 1.0)
    dinv_ref[...] = dinv
    y1_ref[...] = _mm(x_ref[...], w1_ref[...]) * dinv


def _tc2_body(s1a_ref, s1b_ref, y1_ref, dinv_ref, b1_ref, w2_ref, y2_ref):
    dinv = dinv_ref[...]
    h1 = jnp.maximum(dinv * (s1a_ref[...] + s1b_ref[...] + y1_ref[...])
                     + b1_ref[...], 0.0)
    y2_ref[...] = _mm(h1, w2_ref[...]) * dinv


def _tc3_body(s2a_ref, s2b_ref, y2_ref, dinv_ref, b2_ref, wlt_ref, bl_ref,
              h2_ref, out_ref):
    dinv = dinv_ref[...]
    h2 = jnp.maximum(dinv * (s2a_ref[...] + s2b_ref[...] + y2_ref[...])
                     + b2_ref[...], 0.0)
    h2_ref[...] = h2
    out_ref[...] = _mm(h2, wlt_ref[...]) + bl_ref[...]


def kernel(x, edge_index, W1, b1, W2, b2, Wl, bl):
    n, d_in = x.shape
    d_hid = W1.shape[1]
    d_out = W2.shape[1]
    e = edge_index.shape[1]
    epw = e // NW
    nchunk = epw // K
    n_pad = ((n + NS * K - 1) // (NS * K)) * (NS * K)

    src3 = edge_index[0].reshape(NW, nchunk, K)
    dst3 = edge_index[1].reshape(NW, nchunk, K)

    deg_p = _make_deg_kernel(n_pad, nchunk)(dst3)
    deg0 = deg_p[0, :n].reshape(n, 1)
    deg1 = deg_p[1, :n].reshape(n, 1)

    dinv, y1 = pl.pallas_call(
        _tc1_body,
        out_shape=[jax.ShapeDtypeStruct((n, 1), jnp.float32),
                   jax.ShapeDtypeStruct((n, d_hid), jnp.float32)],
    )(x, W1, deg0, deg1)

    s1_p = _make_agg_kernel(n_pad, nchunk, d_hid, 5)(y1, src3, dst3)

    y2 = pl.pallas_call(
        _tc2_body,
        out_shape=jax.ShapeDtypeStruct((n, d_out), jnp.float32),
    )(s1_p[0, :n], s1_p[1, :n], y1, dinv, b1.reshape(1, d_hid), W2)

    s2_p = _make_agg_kernel(n_pad, nchunk, d_out, 1)(y2, src3, dst3)

    h2, out = pl.pallas_call(
        _tc3_body,
        out_shape=[jax.ShapeDtypeStruct((n, d_out), jnp.float32),
                   jax.ShapeDtypeStruct((n, d_out), jnp.float32)],
    )(s2_p[0, :n], s2_p[1, :n], y2, dinv, b2.reshape(1, d_out), Wl.T,
      bl.reshape(1, d_out))

    return (h2, out)


# speedup vs baseline: 31.6203x; 1.0446x over previous
import functools

import jax
import jax.numpy as jnp
from jax import lax
from jax.experimental import pallas as pl
from jax.experimental.pallas import tpu as pltpu
from jax.experimental.pallas import tpu_sc as plsc

NC = 2
NS = 16
NW = NC * NS
K = 80


def _flat_wid():
    return lax.axis_index("s") * NC + lax.axis_index("c")


def _make_deg_kernel(n_pad, nchunk):
    mesh = plsc.VectorSubcoreMesh(core_axis_name="c", subcore_axis_name="s")
    rps = n_pad // NS

    @functools.partial(
        pl.kernel,
        out_type=jax.ShapeDtypeStruct((NC, n_pad), jnp.float32),
        mesh=mesh,
        scratch_types=[
            pltpu.VMEM_SHARED((n_pad,), jnp.float32),
            pltpu.VMEM((nchunk, K), jnp.int32),
            pltpu.VMEM((K,), jnp.float32),
            pltpu.VMEM((rps,), jnp.float32),
        ],
    )
    def k(dst_hbm, out_hbm, acc, dstv, ones_v, zbuf):
        c = lax.axis_index("c")
        s = lax.axis_index("s")
        wid = _flat_wid()

        for i in range(K // 16):
            ones_v[pl.ds(16 * i, 16)] = jnp.ones((16,), jnp.float32)

        def zfill(i, _):
            zbuf[pl.ds(16 * i, 16)] = jnp.zeros((16,), jnp.float32)
            return 0
        lax.fori_loop(0, rps // 16, zfill, 0)
        pltpu.sync_copy(zbuf, acc.at[pl.ds(s * rps, rps)])

        pltpu.sync_copy(dst_hbm.at[wid], dstv)
        plsc.subcore_barrier()

        def body(ci, _):
            pltpu.sync_copy(ones_v, acc.at[dstv.at[ci]], add=True)
            return 0
        lax.fori_loop(0, nchunk, body, 0)

        plsc.subcore_barrier()
        pltpu.sync_copy(acc.at[pl.ds(s * rps, rps)],
                        out_hbm.at[c, pl.ds(s * rps, rps)])

    return k


def _make_agg_kernel(n_pad, nchunk, d, nblk):
    mesh = plsc.VectorSubcoreMesh(core_axis_name="c", subcore_axis_name="s")
    rps = n_pad // NS
    cpb = nchunk // nblk

    @functools.partial(
        pl.kernel,
        out_type=jax.ShapeDtypeStruct((NC, n_pad, d), jnp.float32),
        mesh=mesh,
        scratch_types=[
            pltpu.VMEM_SHARED((n_pad, d), jnp.float32),
            pltpu.VMEM((cpb, K), jnp.int32),
            pltpu.VMEM((cpb, K), jnp.int32),
            pltpu.VMEM((K, d), jnp.float32),
            pltpu.VMEM((K, d), jnp.float32),
            pltpu.SemaphoreType.DMA,
            pltpu.SemaphoreType.DMA,
        ],
        compiler_params=pltpu.CompilerParams(use_tc_tiling_on_sc=False),
    )
    def k(y_hbm, src_hbm, dst_hbm, out_hbm, acc, srcv, dstv, rows0, rows1,
          gsem0, gsem1):
        c = lax.axis_index("c")
        s = lax.axis_index("s")
        wid = _flat_wid()
        bufs = ((rows0, gsem0), (rows1, gsem1))

        def zfill(i, _):
            for j in range(d // 16):
                rows0[i, pl.ds(16 * j, 16)] = jnp.zeros((16,), jnp.float32)
            return 0
        lax.fori_loop(0, K, zfill, 0)
        for t in range(rps // K):
            pltpu.sync_copy(rows0, acc.at[pl.ds(s * rps + t * K, K), :])
        plsc.subcore_barrier()

        npairs = cpb // 2
        for blk in range(nblk):
            pltpu.sync_copy(src_hbm.at[wid, pl.ds(blk * cpb, cpb), :], srcv)
            pltpu.sync_copy(dst_hbm.at[wid, pl.ds(blk * cpb, cpb), :], dstv)
            pltpu.async_copy(y_hbm.at[srcv.at[0]], rows0, gsem0)
            pltpu.async_copy(y_hbm.at[srcv.at[1]], rows1, gsem1)

            def body(g, _):
                for b, (rb, gs) in enumerate(bufs):
                    ci = 2 * g + b
                    pltpu.make_async_copy(y_hbm.at[srcv.at[ci]], rb, gs).wait()
                    pltpu.sync_copy(rb, acc.at[dstv.at[ci]], add=True)
                    pltpu.async_copy(y_hbm.at[srcv.at[ci + 2]], rb, gs)
                return 0
            lax.fori_loop(0, npairs - 1, body, 0)

            for b, (rb, gs) in enumerate(bufs):
                ci = 2 * (npairs - 1) + b
                pltpu.make_async_copy(y_hbm.at[srcv.at[ci]], rb, gs).wait()
                pltpu.sync_copy(rb, acc.at[dstv.at[ci]], add=True)
                if ci + 2 < cpb:
                    pltpu.async_copy(y_hbm.at[srcv.at[ci + 2]], rb, gs)
            if cpb % 2:
                ci = cpb - 1
                rb, gs = bufs[ci % 2]
                pltpu.make_async_copy(y_hbm.at[srcv.at[ci]], rb, gs).wait()
                pltpu.sync_copy(rb, acc.at[dstv.at[ci]], add=True)

        plsc.subcore_barrier()
        pltpu.sync_copy(acc.at[pl.ds(s * rps, rps), :],
                        out_hbm.at[c, pl.ds(s * rps, rps), :])

    return k


def _mm(a, b):
    return jnp.dot(a, b, preferred_element_type=jnp.float32,
                   precision=lax.Precision.HIGHEST)


def _tc1_body(x_ref, w1_ref, d0_ref, d1_ref, dinv_ref, y1_ref):
    dinv = lax.rsqrt(d0_ref[...] + d1_ref[...] + 1.0)
    dinv_ref[...] = dinv
    y1_ref[...] = _mm(x_ref[...], w1_ref[...]) * dinv


def _tc2_body(n, sp_ref, y1_ref, dinv_ref, b1_ref, w2_ref, y2_ref):
    dinv = dinv_ref[...]
    s1 = sp_ref[0, :n, :] + sp_ref[1, :n, :]
    h1 = jnp.maximum(dinv * (s1 + y1_ref[...]) + b1_ref[...], 0.0)
    y2_ref[...] = _mm(h1, w2_ref[...]) * dinv


def _tc3_body(n, sp_ref, y2_ref, dinv_ref, b2_ref, wlt_ref, bl_ref,
              h2_ref, out_ref):
    dinv = dinv_ref[...]
    s2 = sp_ref[0, :n, :] + sp_ref[1, :n, :]
    h2 = jnp.maximum(dinv * (s2 + y2_ref[...]) + b2_ref[...], 0.0)
    h2_ref[...] = h2
    out_ref[...] = _mm(h2, wlt_ref[...]) + bl_ref[...]


def kernel(x, edge_index, W1, b1, W2, b2, Wl, bl):
    n, d_in = x.shape
    d_hid = W1.shape[1]
    d_out = W2.shape[1]
    e = edge_index.shape[1]
    epw = e // NW
    nchunk = epw // K
    n_pad = ((n + NS * K - 1) // (NS * K)) * (NS * K)

    src3 = edge_index[0].reshape(NW, nchunk, K)
    dst3 = edge_index[1].reshape(NW, nchunk, K)

    deg_p = _make_deg_kernel(n_pad, nchunk)(dst3)
    deg0 = deg_p[0, :n].reshape(n, 1)
    deg1 = deg_p[1, :n].reshape(n, 1)

    dinv, y1 = pl.pallas_call(
        _tc1_body,
        out_shape=[jax.ShapeDtypeStruct((n, 1), jnp.float32),
                   jax.ShapeDtypeStruct((n, d_hid), jnp.float32)],
    )(x, W1, deg0, deg1)

    s1_p = _make_agg_kernel(n_pad, nchunk, d_hid, 5)(y1, src3, dst3)

    y2 = pl.pallas_call(
        functools.partial(_tc2_body, n),
        out_shape=jax.ShapeDtypeStruct((n, d_out), jnp.float32),
    )(s1_p, y1, dinv, b1.reshape(1, d_hid), W2)

    s2_p = _make_agg_kernel(n_pad, nchunk, d_out, 1)(y2, src3, dst3)

    h2, out = pl.pallas_call(
        functools.partial(_tc3_body, n),
        out_shape=[jax.ShapeDtypeStruct((n, d_out), jnp.float32),
                   jax.ShapeDtypeStruct((n, d_out), jnp.float32)],
    )(s2_p, y2, dinv, b2.reshape(1, d_out), Wl.T, bl.reshape(1, d_out))

    return (h2, out)
